# baseline (speedup 1.0000x reference)
import functools

import jax
import jax.numpy as jnp
from jax import lax
from jax.experimental import pallas as pl
from jax.experimental.pallas import tpu as pltpu
from jax.experimental.pallas import tpu_sc as plsc

DIM = 768
MAX_LEN = 8192
BATCH = 4
SEQ = 8192

NUM_CORES = 2
NUM_SUBCORES = 16
NUM_WORKERS = NUM_CORES * NUM_SUBCORES

B_TOTAL = BATCH * SEQ
B_PER_W = B_TOTAL // NUM_WORKERS
CHUNK = 64
N_CHUNKS = B_PER_W // CHUNK
NBUF = 2
N_GROUPS = N_CHUNKS // NBUF


def _sc_gather(table, idx_flat):
    mesh = plsc.VectorSubcoreMesh(core_axis_name="c", subcore_axis_name="s")

    @functools.partial(
        pl.kernel,
        mesh=mesh,
        out_type=jax.ShapeDtypeStruct((B_TOTAL, DIM), jnp.float32),
        scratch_types=[
            pltpu.VMEM((B_PER_W,), jnp.int32),
        ]
        + [pltpu.VMEM((CHUNK, DIM), jnp.float32)] * NBUF
        + [pltpu.SemaphoreType.DMA] * (2 * NBUF),
    )
    def k(table_hbm, idx_hbm, out_hbm, idx_v, *bufs):
        rows = list(bufs[:NBUF])
        gsem = list(bufs[NBUF : 2 * NBUF])
        ssem = list(bufs[2 * NBUF :])

        wid = lax.axis_index("s") * NUM_CORES + lax.axis_index("c")
        base = wid * B_PER_W
        pltpu.sync_copy(idx_hbm.at[pl.ds(base, B_PER_W)], idx_v)

        def gather_start(b, ci):
            return pltpu.async_copy(
                table_hbm.at[idx_v.at[pl.ds(ci * CHUNK, CHUNK)]], rows[b], gsem[b]
            )

        def gather_wait(b):
            pltpu.make_async_copy(
                table_hbm.at[idx_v.at[pl.ds(0, CHUNK)]], rows[b], gsem[b]
            ).wait()

        def store_start(b, ci):
            return pltpu.async_copy(
                rows[b], out_hbm.at[pl.ds(base + ci * CHUNK, CHUNK)], ssem[b]
            )

        for b in range(NBUF):
            gather_start(b, b)

        @pl.loop(0, N_GROUPS - 1)
        def _(j):
            cb = j * NBUF
            for b in range(NBUF):
                ci = cb + b
                gather_wait(b)
                store_start(b, ci).wait()
                gather_start(b, ci + NBUF)

        last = (N_GROUPS - 1) * NBUF
        s = []
        for b in range(NBUF):
            gather_wait(b)
            s.append(store_start(b, last + b))
        for h in s:
            h.wait()

    return k(table, idx_flat)


@jax.jit
def kernel(position_ids, table):
    idx_flat = position_ids.reshape(B_TOTAL).astype(jnp.int32)
    out = _sc_gather(table, idx_flat)
    return out.reshape(BATCH, SEQ, DIM)


# speedup vs baseline: 1.0111x; 1.0111x over previous
import functools

import jax
import jax.numpy as jnp
from jax import lax
from jax.experimental import pallas as pl
from jax.experimental.pallas import tpu as pltpu
from jax.experimental.pallas import tpu_sc as plsc

DIM = 768
MAX_LEN = 8192
BATCH = 4
SEQ = 8192

NUM_CORES = 2
NUM_SUBCORES = 16
NUM_WORKERS = NUM_CORES * NUM_SUBCORES

B_TOTAL = BATCH * SEQ
B_PER_W = B_TOTAL // NUM_WORKERS
CHUNK = 32
N_CHUNKS = B_PER_W // CHUNK
NBUF = 4
N_GROUPS = N_CHUNKS // NBUF


def _sc_gather(table, idx_flat):
    mesh = plsc.VectorSubcoreMesh(core_axis_name="c", subcore_axis_name="s")

    @functools.partial(
        pl.kernel,
        mesh=mesh,
        out_type=jax.ShapeDtypeStruct((BATCH, SEQ, DIM), jnp.float32),
        scratch_types=[
            pltpu.VMEM((B_PER_W,), jnp.int32),
        ]
        + [pltpu.VMEM((CHUNK, DIM), jnp.float32)] * NBUF
        + [pltpu.SemaphoreType.DMA] * (2 * NBUF),
    )
    def k(table_hbm, idx_hbm, out_hbm, idx_v, *bufs):
        rows = list(bufs[:NBUF])
        gsem = list(bufs[NBUF : 2 * NBUF])
        ssem = list(bufs[2 * NBUF :])

        wid = lax.axis_index("s") * NUM_CORES + lax.axis_index("c")
        w_per_b = SEQ // B_PER_W
        bi = wid // w_per_b
        col = (wid % w_per_b) * B_PER_W
        pltpu.sync_copy(idx_hbm.at[bi, pl.ds(col, B_PER_W)], idx_v)

        def gather_start(b, ci):
            return pltpu.async_copy(
                table_hbm.at[idx_v.at[pl.ds(ci * CHUNK, CHUNK)]], rows[b], gsem[b]
            )

        def gather_wait(b):
            pltpu.make_async_copy(
                table_hbm.at[idx_v.at[pl.ds(0, CHUNK)]], rows[b], gsem[b]
            ).wait()

        def store_start(b, ci):
            return pltpu.async_copy(
                rows[b], out_hbm.at[bi, pl.ds(col + ci * CHUNK, CHUNK)], ssem[b]
            )

        for b in range(NBUF):
            gather_start(b, b)

        @pl.loop(0, N_GROUPS - 1)
        def _(j):
            cb = j * NBUF
            for b in range(NBUF):
                ci = cb + b
                gather_wait(b)
                store_start(b, ci).wait()
                gather_start(b, ci + NBUF)

        last = (N_GROUPS - 1) * NBUF
        s = []
        for b in range(NBUF):
            gather_wait(b)
            s.append(store_start(b, last + b))
        for h in s:
            h.wait()

    return k(table, idx_flat)


@jax.jit
def kernel(position_ids, table):
    return _sc_gather(table, position_ids)


# speedup vs baseline: 1.0135x; 1.0024x over previous
import functools

import jax
import jax.numpy as jnp
from jax import lax
from jax.experimental import pallas as pl
from jax.experimental.pallas import tpu as pltpu
from jax.experimental.pallas import tpu_sc as plsc

DIM = 768
MAX_LEN = 8192
BATCH = 4
SEQ = 8192

NUM_CORES = 2
NUM_SUBCORES = 16
NUM_WORKERS = NUM_CORES * NUM_SUBCORES

B_TOTAL = BATCH * SEQ
B_PER_W = B_TOTAL // NUM_WORKERS
CHUNK = 32
N_CHUNKS = B_PER_W // CHUNK
NBUF = 4
N_GROUPS = N_CHUNKS // NBUF


def _sc_gather(table, idx_flat):
    mesh = plsc.VectorSubcoreMesh(core_axis_name="c", subcore_axis_name="s")

    @functools.partial(
        pl.kernel,
        mesh=mesh,
        out_type=jax.ShapeDtypeStruct((BATCH, SEQ, DIM), jnp.float32),
        scratch_types=[
            pltpu.VMEM((B_PER_W,), jnp.int32),
        ]
        + [pltpu.VMEM((CHUNK, DIM), jnp.float32)] * NBUF
        + [pltpu.SemaphoreType.DMA] * (2 * NBUF),
    )
    def k(table_hbm, idx_hbm, out_hbm, idx_v, *bufs):
        rows = list(bufs[:NBUF])
        gsem = list(bufs[NBUF : 2 * NBUF])
        ssem = list(bufs[2 * NBUF :])

        wid = lax.axis_index("s") * NUM_CORES + lax.axis_index("c")
        w_per_b = SEQ // B_PER_W
        bi = wid // w_per_b
        col = (wid % w_per_b) * B_PER_W
        pltpu.sync_copy(idx_hbm.at[bi, pl.ds(col, B_PER_W)], idx_v)

        def gather_start(b, ci):
            return pltpu.async_copy(
                table_hbm.at[idx_v.at[pl.ds(ci * CHUNK, CHUNK)]], rows[b], gsem[b]
            )

        def gather_wait(b):
            pltpu.make_async_copy(
                table_hbm.at[idx_v.at[pl.ds(0, CHUNK)]], rows[b], gsem[b]
            ).wait()

        def store_start(b, ci):
            return pltpu.async_copy(
                rows[b], out_hbm.at[bi, pl.ds(col + ci * CHUNK, CHUNK)], ssem[b]
            )

        @pl.loop(0, N_GROUPS + 1)
        def _(j):
            for b in range(NBUF):

                @pl.when(j > 0)
                def _():
                    gather_wait(b)
                    store_start(b, (j - 1) * NBUF + b).wait()

                @pl.when(j < N_GROUPS)
                def _():
                    gather_start(b, j * NBUF + b)

    return k(table, idx_flat)


@jax.jit
def kernel(position_ids, table):
    return _sc_gather(table, position_ids)
